# Initial kernel scaffold; baseline (speedup 1.0000x reference)
import functools

import jax
import jax.numpy as jnp
from jax import lax
from jax.experimental import pallas as pl
from jax.experimental.pallas import tpu as pltpu
from jax.experimental.pallas import tpu_sc as plsc

N = 10000
E = 320000
D = 128
C = 7

NC = 2
NS = 16
NW = NC * NS
L = 16

NLOC = 320
NPAD = NW * NLOC
CH = 3200
G = 128
CPAD = CH + 2 * G
NEG = float("-inf")


def _sc_body(x_hbm, src_hbm, dst_hbm, out_hbm,
             agg_v, src_v, dst_v, cs_v, cd_v, rows_v, sem):
  wid = lax.axis_index("s") * NC + lax.axis_index("c")
  lo = wid * NLOC

  neg = jnp.full((L,), NEG, dtype=jnp.float32)

  def init_body(i, _):
    for j in range(D // L):
      agg_v[i, pl.ds(j * L, L)] = neg
    return 0

  lax.fori_loop(0, NLOC, init_body, 0)

  lane = lax.iota(jnp.int32, L)
  lo_v = jnp.full((L,), lo, dtype=jnp.int32)
  nloc_u = jnp.full((L,), NLOC, dtype=jnp.uint32)

  def chunk_body(c, _):
    base = c * CH
    pltpu.sync_copy(src_hbm.at[pl.ds(base, CH)], src_v)
    pltpu.sync_copy(dst_hbm.at[pl.ds(base, CH)], dst_v)

    def filt_body(i, cnt_v):
      s = src_v[pl.ds(i * L, L)]
      d = dst_v[pl.ds(i * L, L)]
      dl = d - lo_v
      m = lax.bitcast_convert_type(dl, jnp.uint32) < nloc_u
      mi = m.astype(jnp.int32)
      rank = plsc.cumsum(mi) - mi
      pos = cnt_v + rank
      plsc.store_scatter(cs_v, [pos], s, mask=m)
      plsc.store_scatter(cd_v, [pos], dl, mask=m)
      return cnt_v + plsc.all_reduce_population_count(m)

    cnt_v = lax.fori_loop(0, CH // L, filt_body,
                          jnp.zeros((L,), jnp.int32), unroll=4)
    cnt = lax.reduce_max(cnt_v, axes=(0,))

    f16 = (cnt // L) * L
    rem = cnt - f16
    tail = cs_v[pl.ds(f16, L)]
    cs_v[pl.ds(f16, L)] = jnp.where(lane < rem, tail, lo_v)
    for t in range(G // L):
      cs_v[pl.ds(f16 + (t + 1) * L, L)] = lo_v

    def blk_body(k, _):
      pltpu.async_copy(x_hbm.at[cs_v.at[pl.ds(k * G, G)]], rows_v, sem).wait()
      nmax = jnp.minimum(cnt - k * G, G)

      def edge_body(e, _):
        dl = cd_v[k * G + e]
        for j in range(D // L):
          sl = pl.ds(j * L, L)
          agg_v[dl, sl] = jnp.maximum(agg_v[dl, sl], rows_v[e, sl])
        return 0

      lax.fori_loop(0, nmax, edge_body, 0)
      return 0

    nb = (cnt + G - 1) // G
    lax.fori_loop(0, nb, blk_body, 0)
    return 0

  lax.fori_loop(0, E // CH, chunk_body, 0)

  pltpu.sync_copy(agg_v, out_hbm.at[pl.ds(lo, NLOC)])


_sc_mesh = plsc.VectorSubcoreMesh(core_axis_name="c", subcore_axis_name="s",
                                  num_cores=NC, num_subcores=NS)

_sc_call = functools.partial(
    pl.kernel,
    mesh=_sc_mesh,
    out_type=jax.ShapeDtypeStruct((NPAD, D), jnp.float32),
    scratch_types=[
        pltpu.VMEM((NLOC, D), jnp.float32),
        pltpu.VMEM((CH,), jnp.int32),
        pltpu.VMEM((CH,), jnp.int32),
        pltpu.VMEM((CPAD,), jnp.int32),
        pltpu.VMEM((CPAD,), jnp.int32),
        pltpu.VMEM((G, D), jnp.float32),
        pltpu.SemaphoreType.DMA,
    ],
)(_sc_body)


def _tc_body(agg_ref, x_ref, wl_ref, wr_ref, b_ref, out_ref):
  agg = agg_ref[...][:N]
  agg = jnp.where(jnp.isfinite(agg), agg, 0.0)
  logits = (
      jax.lax.dot_general(agg, wl_ref[...], (((1,), (0,)), ((), ())),
                          preferred_element_type=jnp.float32)
      + jax.lax.dot_general(x_ref[...], wr_ref[...], (((1,), (0,)), ((), ())),
                            preferred_element_type=jnp.float32)
      + b_ref[...]
  )
  m = jnp.max(logits, axis=1, keepdims=True)
  s = logits - m
  out_ref[...] = s - jnp.log(jnp.sum(jnp.exp(s), axis=1, keepdims=True))


_tc_call = pl.pallas_call(
    _tc_body,
    out_shape=jax.ShapeDtypeStruct((N, C), jnp.float32),
)


@jax.jit
def kernel(x, edge_index, W_l, b_l, W_r):
  src = edge_index[0]
  dst = edge_index[1]
  agg = _sc_call(x, src, dst)
  return _tc_call(agg, x, W_l.T, W_r.T, b_l.reshape(1, C))


# speedup vs baseline: 1.6773x; 1.6773x over previous
import functools

import jax
import jax.numpy as jnp
from jax import lax
from jax.experimental import pallas as pl
from jax.experimental.pallas import tpu as pltpu
from jax.experimental.pallas import tpu_sc as plsc

N = 10000
E = 320000
D = 128
C = 7

NC = 2
NS = 16
NW = NC * NS
L = 16

NLOC = 320
NPAD = NW * NLOC
CH = 3200
G = 128
CPAD = CH + 2 * G
NEG = float("-inf")


def _sc_body(x_hbm, src_hbm, dst_hbm, out_hbm,
             agg_v, src_v, dst_v, cs_v, cd_v, rows_v, sem):
  wid = lax.axis_index("s") * NC + lax.axis_index("c")
  lo = wid * NLOC

  neg = jnp.full((L,), NEG, dtype=jnp.float32)

  def init_body(i, _):
    for j in range(D // L):
      agg_v[i, pl.ds(j * L, L)] = neg
    return 0

  lax.fori_loop(0, NLOC + 1, init_body, 0)

  lane = lax.iota(jnp.int32, L)
  lo_v = jnp.full((L,), lo, dtype=jnp.int32)
  nloc_u = jnp.full((L,), NLOC, dtype=jnp.uint32)
  nloc_i = jnp.full((L,), NLOC, dtype=jnp.int32)

  def chunk_body(c, _):
    base = c * CH
    pltpu.sync_copy(src_hbm.at[pl.ds(base, CH)], src_v)
    pltpu.sync_copy(dst_hbm.at[pl.ds(base, CH)], dst_v)

    def filt_body(i, cnt_v):
      s = src_v[pl.ds(i * L, L)]
      d = dst_v[pl.ds(i * L, L)]
      dl = d - lo_v
      m = lax.bitcast_convert_type(dl, jnp.uint32) < nloc_u
      mi = m.astype(jnp.int32)
      rank = plsc.cumsum(mi) - mi
      pos = cnt_v + rank
      plsc.store_scatter(cs_v, [pos], s, mask=m)
      plsc.store_scatter(cd_v, [pos], dl, mask=m)
      return cnt_v + plsc.all_reduce_population_count(m)

    cnt_v = lax.fori_loop(0, CH // L, filt_body,
                          jnp.zeros((L,), jnp.int32), unroll=4)
    cnt = lax.reduce_max(cnt_v, axes=(0,))

    f16 = (cnt // L) * L
    rem = cnt - f16
    keep = lane < rem
    tail_s = cs_v[pl.ds(f16, L)]
    cs_v[pl.ds(f16, L)] = jnp.where(keep, tail_s, lo_v)
    tail_d = cd_v[pl.ds(f16, L)]
    cd_v[pl.ds(f16, L)] = jnp.where(keep, tail_d, nloc_i)
    for t in range(G // L):
      cs_v[pl.ds(f16 + (t + 1) * L, L)] = lo_v
      cd_v[pl.ds(f16 + (t + 1) * L, L)] = nloc_i

    def blk_body(k, _):
      pltpu.async_copy(x_hbm.at[cs_v.at[pl.ds(k * G, G)]], rows_v, sem).wait()
      ngrp = (jnp.minimum(cnt - k * G, G) + L - 1) // L

      def grp_body(g, _):
        dlv = cd_v[pl.ds(k * G + g * L, L)]
        for t in range(L):
          dl = dlv[t]
          for j in range(D // L):
            sl = pl.ds(j * L, L)
            agg_v[dl, sl] = jnp.maximum(agg_v[dl, sl],
                                        rows_v[g * L + t, sl])
        return 0

      lax.fori_loop(0, ngrp, grp_body, 0)
      return 0

    nb = (cnt + G - 1) // G
    lax.fori_loop(0, nb, blk_body, 0)
    return 0

  lax.fori_loop(0, E // CH, chunk_body, 0)

  pltpu.sync_copy(agg_v.at[pl.ds(0, NLOC)], out_hbm.at[pl.ds(lo, NLOC)])


_sc_mesh = plsc.VectorSubcoreMesh(core_axis_name="c", subcore_axis_name="s",
                                  num_cores=NC, num_subcores=NS)

_sc_call = functools.partial(
    pl.kernel,
    mesh=_sc_mesh,
    out_type=jax.ShapeDtypeStruct((NPAD, D), jnp.float32),
    scratch_types=[
        pltpu.VMEM((NLOC + 1, D), jnp.float32),
        pltpu.VMEM((CH,), jnp.int32),
        pltpu.VMEM((CH,), jnp.int32),
        pltpu.VMEM((CPAD,), jnp.int32),
        pltpu.VMEM((CPAD,), jnp.int32),
        pltpu.VMEM((G, D), jnp.float32),
        pltpu.SemaphoreType.DMA,
    ],
    compiler_params=pltpu.CompilerParams(needs_layout_passes=False),
)(_sc_body)


def _tc_body(agg_ref, x_ref, wl_ref, wr_ref, b_ref, out_ref):
  agg = agg_ref[...][:N]
  agg = jnp.where(jnp.isfinite(agg), agg, 0.0)
  logits = (
      jax.lax.dot_general(agg, wl_ref[...], (((1,), (0,)), ((), ())),
                          preferred_element_type=jnp.float32)
      + jax.lax.dot_general(x_ref[...], wr_ref[...], (((1,), (0,)), ((), ())),
                            preferred_element_type=jnp.float32)
      + b_ref[...]
  )
  m = jnp.max(logits, axis=1, keepdims=True)
  s = logits - m
  out_ref[...] = s - jnp.log(jnp.sum(jnp.exp(s), axis=1, keepdims=True))


_tc_call = pl.pallas_call(
    _tc_body,
    out_shape=jax.ShapeDtypeStruct((N, C), jnp.float32),
)


@jax.jit
def kernel(x, edge_index, W_l, b_l, W_r):
  src = edge_index[0]
  dst = edge_index[1]
  agg = _sc_call(x, src, dst)
  return _tc_call(agg, x, W_l.T, W_r.T, b_l.reshape(1, C))


# speedup vs baseline: 1.8219x; 1.0862x over previous
import functools

import jax
import jax.numpy as jnp
from jax import lax
from jax.experimental import pallas as pl
from jax.experimental.pallas import tpu as pltpu
from jax.experimental.pallas import tpu_sc as plsc

N = 10000
E = 320000
D = 128
C = 7

NC = 2
NS = 16
NW = NC * NS
L = 16

NLOC = 320
NPAD = NW * NLOC
CH = 3200
G = 128
CPAD = CH + 2 * G
NEG = float("-inf")


def _sc_body(x_hbm, src_hbm, dst_hbm, out_hbm,
             agg_v, src_v, dst_v, cs_v, cd_v, rows_v, sem):
  wid = lax.axis_index("s") * NC + lax.axis_index("c")
  lo = wid * NLOC

  neg = jnp.full((L,), NEG, dtype=jnp.float32)

  def init_body(i, _):
    for j in range(D // L):
      agg_v[i, pl.ds(j * L, L)] = neg
    return 0

  lax.fori_loop(0, NLOC + 1, init_body, 0)

  lane = lax.iota(jnp.int32, L)
  lo_v = jnp.full((L,), lo, dtype=jnp.int32)
  nloc_u = jnp.full((L,), NLOC, dtype=jnp.uint32)
  nloc_i = jnp.full((L,), NLOC, dtype=jnp.int32)

  def chunk_body(c, _):
    base = c * CH
    pltpu.sync_copy(src_hbm.at[pl.ds(base, CH)], src_v)
    pltpu.sync_copy(dst_hbm.at[pl.ds(base, CH)], dst_v)

    def filt_body(i, cnt_v):
      s = src_v[pl.ds(i * L, L)]
      d = dst_v[pl.ds(i * L, L)]
      dl = d - lo_v
      m = lax.bitcast_convert_type(dl, jnp.uint32) < nloc_u
      mi = m.astype(jnp.int32)
      rank = plsc.cumsum(mi) - mi
      pos = cnt_v + rank
      plsc.store_scatter(cs_v, [pos], s, mask=m)
      plsc.store_scatter(cd_v, [pos], dl, mask=m)
      return cnt_v + plsc.all_reduce_population_count(m)

    cnt_v = lax.fori_loop(0, CH // L, filt_body,
                          jnp.zeros((L,), jnp.int32), unroll=4)
    cnt = lax.reduce_max(cnt_v, axes=(0,))

    f16 = (cnt // L) * L
    rem = cnt - f16
    keep = lane < rem
    tail_s = cs_v[pl.ds(f16, L)]
    cs_v[pl.ds(f16, L)] = jnp.where(keep, tail_s, lo_v)
    tail_d = cd_v[pl.ds(f16, L)]
    cd_v[pl.ds(f16, L)] = jnp.where(keep, tail_d, nloc_i)
    for t in range(G // L):
      cs_v[pl.ds(f16 + (t + 1) * L, L)] = lo_v
      cd_v[pl.ds(f16 + (t + 1) * L, L)] = nloc_i

    def blk_body(k, _):
      pltpu.async_copy(x_hbm.at[cs_v.at[pl.ds(k * G, G)]], rows_v, sem).wait()
      ngrp = (jnp.minimum(cnt - k * G, G) + L - 1) // L

      def grp_body(g, _):
        dlv = cd_v[pl.ds(k * G + g * L, L)]
        dls = [dlv[t] for t in range(L)]
        for t in range(L):
          dl = dls[t]
          rvals = [rows_v[g * L + t, pl.ds(j * L, L)] for j in range(D // L)]
          avals = [agg_v[dl, pl.ds(j * L, L)] for j in range(D // L)]
          for j in range(D // L):
            agg_v[dl, pl.ds(j * L, L)] = jnp.maximum(avals[j], rvals[j])
        return 0

      lax.fori_loop(0, ngrp, grp_body, 0)
      return 0

    nb = (cnt + G - 1) // G
    lax.fori_loop(0, nb, blk_body, 0)
    return 0

  lax.fori_loop(0, E // CH, chunk_body, 0)

  pltpu.sync_copy(agg_v.at[pl.ds(0, NLOC)], out_hbm.at[pl.ds(lo, NLOC)])


_sc_mesh = plsc.VectorSubcoreMesh(core_axis_name="c", subcore_axis_name="s",
                                  num_cores=NC, num_subcores=NS)

_sc_call = functools.partial(
    pl.kernel,
    mesh=_sc_mesh,
    out_type=jax.ShapeDtypeStruct((NPAD, D), jnp.float32),
    scratch_types=[
        pltpu.VMEM((NLOC + 1, D), jnp.float32),
        pltpu.VMEM((CH,), jnp.int32),
        pltpu.VMEM((CH,), jnp.int32),
        pltpu.VMEM((CPAD,), jnp.int32),
        pltpu.VMEM((CPAD,), jnp.int32),
        pltpu.VMEM((G, D), jnp.float32),
        pltpu.SemaphoreType.DMA,
    ],
    compiler_params=pltpu.CompilerParams(needs_layout_passes=False),
)(_sc_body)


def _tc_body(agg_ref, x_ref, wl_ref, wr_ref, b_ref, out_ref):
  agg = agg_ref[...][:N]
  agg = jnp.where(jnp.isfinite(agg), agg, 0.0)
  logits = (
      jax.lax.dot_general(agg, wl_ref[...], (((1,), (0,)), ((), ())),
                          preferred_element_type=jnp.float32)
      + jax.lax.dot_general(x_ref[...], wr_ref[...], (((1,), (0,)), ((), ())),
                            preferred_element_type=jnp.float32)
      + b_ref[...]
  )
  m = jnp.max(logits, axis=1, keepdims=True)
  s = logits - m
  out_ref[...] = s - jnp.log(jnp.sum(jnp.exp(s), axis=1, keepdims=True))


_tc_call = pl.pallas_call(
    _tc_body,
    out_shape=jax.ShapeDtypeStruct((N, C), jnp.float32),
)


@jax.jit
def kernel(x, edge_index, W_l, b_l, W_r):
  src = edge_index[0]
  dst = edge_index[1]
  agg = _sc_call(x, src, dst)
  return _tc_call(agg, x, W_l.T, W_r.T, b_l.reshape(1, C))


# speedup vs baseline: 1.9394x; 1.0645x over previous
import functools

import jax
import jax.numpy as jnp
from jax import lax
from jax.experimental import pallas as pl
from jax.experimental.pallas import tpu as pltpu
from jax.experimental.pallas import tpu_sc as plsc

N = 10000
E = 320000
D = 128
C = 7

NC = 2
NS = 16
NW = NC * NS
L = 16

NLOC = 320
NPAD = NW * NLOC
CH = 6400
G = 256
CPAD = CH + 2 * G
NEG = float("-inf")


def _sc_body(x_hbm, src_hbm, dst_hbm, out_hbm,
             agg_v, src_v, dst_v, cs_v, cd_v, rows_v, sem):
  wid = lax.axis_index("s") * NC + lax.axis_index("c")
  lo = wid * NLOC

  neg = jnp.full((L,), NEG, dtype=jnp.float32)

  def init_body(i, _):
    for j in range(D // L):
      agg_v[i, pl.ds(j * L, L)] = neg
    return 0

  lax.fori_loop(0, NLOC + 1, init_body, 0)

  lane = lax.iota(jnp.int32, L)
  lo_v = jnp.full((L,), lo, dtype=jnp.int32)
  nloc_u = jnp.full((L,), NLOC, dtype=jnp.uint32)
  nloc_i = jnp.full((L,), NLOC, dtype=jnp.int32)

  def chunk_body(c, _):
    base = c * CH
    pltpu.sync_copy(src_hbm.at[pl.ds(base, CH)], src_v)
    pltpu.sync_copy(dst_hbm.at[pl.ds(base, CH)], dst_v)

    def filt_body(i, cnt_v):
      s = src_v[pl.ds(i * L, L)]
      d = dst_v[pl.ds(i * L, L)]
      dl = d - lo_v
      m = lax.bitcast_convert_type(dl, jnp.uint32) < nloc_u
      mi = m.astype(jnp.int32)
      rank = plsc.cumsum(mi) - mi
      pos = cnt_v + rank
      plsc.store_scatter(cs_v, [pos], s, mask=m)
      plsc.store_scatter(cd_v, [pos], dl, mask=m)
      return cnt_v + plsc.all_reduce_population_count(m)

    cnt_v = lax.fori_loop(0, CH // L, filt_body,
                          jnp.zeros((L,), jnp.int32), unroll=4)
    cnt = lax.reduce_max(cnt_v, axes=(0,))

    f16 = (cnt // L) * L
    rem = cnt - f16
    keep = lane < rem
    tail_s = cs_v[pl.ds(f16, L)]
    cs_v[pl.ds(f16, L)] = jnp.where(keep, tail_s, lo_v)
    tail_d = cd_v[pl.ds(f16, L)]
    cd_v[pl.ds(f16, L)] = jnp.where(keep, tail_d, nloc_i)
    for t in range(G // L):
      cs_v[pl.ds(f16 + (t + 1) * L, L)] = lo_v
      cd_v[pl.ds(f16 + (t + 1) * L, L)] = nloc_i

    def blk_body(k, _):
      pltpu.async_copy(x_hbm.at[cs_v.at[pl.ds(k * G, G)]], rows_v, sem).wait()
      ngrp = (jnp.minimum(cnt - k * G, G) + L - 1) // L

      def grp_body(g, _):
        dlv = cd_v[pl.ds(k * G + g * L, L)]
        dls = [dlv[t] for t in range(L)]
        for t in range(L):
          dl = dls[t]
          rvals = [rows_v[g * L + t, pl.ds(j * L, L)] for j in range(D // L)]
          avals = [agg_v[dl, pl.ds(j * L, L)] for j in range(D // L)]
          for j in range(D // L):
            agg_v[dl, pl.ds(j * L, L)] = jnp.maximum(avals[j], rvals[j])
        return 0

      lax.fori_loop(0, ngrp, grp_body, 0)
      return 0

    nb = (cnt + G - 1) // G
    lax.fori_loop(0, nb, blk_body, 0)
    return 0

  lax.fori_loop(0, E // CH, chunk_body, 0)

  pltpu.sync_copy(agg_v.at[pl.ds(0, NLOC)], out_hbm.at[pl.ds(lo, NLOC)])


_sc_mesh = plsc.VectorSubcoreMesh(core_axis_name="c", subcore_axis_name="s",
                                  num_cores=NC, num_subcores=NS)

_sc_call = functools.partial(
    pl.kernel,
    mesh=_sc_mesh,
    out_type=jax.ShapeDtypeStruct((NPAD, D), jnp.float32),
    scratch_types=[
        pltpu.VMEM((NLOC + 1, D), jnp.float32),
        pltpu.VMEM((CH,), jnp.int32),
        pltpu.VMEM((CH,), jnp.int32),
        pltpu.VMEM((CPAD,), jnp.int32),
        pltpu.VMEM((CPAD,), jnp.int32),
        pltpu.VMEM((G, D), jnp.float32),
        pltpu.SemaphoreType.DMA,
    ],
    compiler_params=pltpu.CompilerParams(needs_layout_passes=False),
)(_sc_body)


def _tc_body(agg_ref, x_ref, wl_ref, wr_ref, b_ref, out_ref):
  agg = agg_ref[...][:N]
  agg = jnp.where(jnp.isfinite(agg), agg, 0.0)
  logits = (
      jax.lax.dot_general(agg, wl_ref[...], (((1,), (0,)), ((), ())),
                          preferred_element_type=jnp.float32)
      + jax.lax.dot_general(x_ref[...], wr_ref[...], (((1,), (0,)), ((), ())),
                            preferred_element_type=jnp.float32)
      + b_ref[...]
  )
  m = jnp.max(logits, axis=1, keepdims=True)
  s = logits - m
  out_ref[...] = s - jnp.log(jnp.sum(jnp.exp(s), axis=1, keepdims=True))


_tc_call = pl.pallas_call(
    _tc_body,
    out_shape=jax.ShapeDtypeStruct((N, C), jnp.float32),
)


@jax.jit
def kernel(x, edge_index, W_l, b_l, W_r):
  src = edge_index[0]
  dst = edge_index[1]
  agg = _sc_call(x, src, dst)
  return _tc_call(agg, x, W_l.T, W_r.T, b_l.reshape(1, C))


# speedup vs baseline: 2.2635x; 1.1671x over previous
import functools

import jax
import jax.numpy as jnp
from jax import lax
from jax.experimental import pallas as pl
from jax.experimental.pallas import tpu as pltpu
from jax.experimental.pallas import tpu_sc as plsc

N = 10000
E = 320000
D = 128
C = 7

NC = 2
NS = 16
NW = NC * NS
L = 16

NLOC = 320
NPAD = NW * NLOC
CH = 6400
G = 256
CPAD = CH + 2 * G
NEG = float("-inf")


def _sc_body(x_hbm, src_hbm, dst_hbm, out_hbm,
             agg_v, src_v, dst_v, cs_v, cd_v, rows_v, sem):
  wid = lax.axis_index("s") * NC + lax.axis_index("c")
  lo = wid * NLOC

  neg = jnp.full((L,), NEG, dtype=jnp.float32)

  def init_body(i, _):
    for j in range(D // L):
      agg_v[i, pl.ds(j * L, L)] = neg
    return 0

  lax.fori_loop(0, NLOC + 1, init_body, 0)

  lane = lax.iota(jnp.int32, L)
  lo_v = jnp.full((L,), lo, dtype=jnp.int32)
  nloc_u = jnp.full((L,), NLOC, dtype=jnp.uint32)
  nloc_i = jnp.full((L,), NLOC, dtype=jnp.int32)

  def chunk_body(c, _):
    base = c * CH
    pltpu.sync_copy(src_hbm.at[pl.ds(base, CH)], src_v)
    pltpu.sync_copy(dst_hbm.at[pl.ds(base, CH)], dst_v)

    FB = 4

    def filt_body(i, cnt_v):
      ss = [src_v[pl.ds((FB * i + k) * L, L)] for k in range(FB)]
      dd = [dst_v[pl.ds((FB * i + k) * L, L)] for k in range(FB)]
      dls = [d - lo_v for d in dd]
      ms = [lax.bitcast_convert_type(dl, jnp.uint32) < nloc_u for dl in dls]
      mis = [m.astype(jnp.int32) for m in ms]
      ranks = [plsc.cumsum(mi) - mi for mi in mis]
      pops = [plsc.all_reduce_population_count(m) for m in ms]
      bases = [cnt_v]
      for k in range(FB - 1):
        bases.append(bases[k] + pops[k])
      for k in range(FB):
        pos = bases[k] + ranks[k]
        plsc.store_scatter(cs_v, [pos], ss[k], mask=ms[k])
        plsc.store_scatter(cd_v, [pos], dls[k], mask=ms[k])
      return bases[FB - 1] + pops[FB - 1]

    cnt_v = lax.fori_loop(0, CH // (FB * L), filt_body,
                          jnp.zeros((L,), jnp.int32), unroll=2)
    cnt = lax.reduce_max(cnt_v, axes=(0,))

    f16 = (cnt // L) * L
    rem = cnt - f16
    keep = lane < rem
    tail_s = cs_v[pl.ds(f16, L)]
    cs_v[pl.ds(f16, L)] = jnp.where(keep, tail_s, lo_v)
    tail_d = cd_v[pl.ds(f16, L)]
    cd_v[pl.ds(f16, L)] = jnp.where(keep, tail_d, nloc_i)
    for t in range(G // L):
      cs_v[pl.ds(f16 + (t + 1) * L, L)] = lo_v
      cd_v[pl.ds(f16 + (t + 1) * L, L)] = nloc_i

    def blk_body(k, _):
      pltpu.async_copy(x_hbm.at[cs_v.at[pl.ds(k * G, G)]], rows_v, sem).wait()
      ngrp = (jnp.minimum(cnt - k * G, G) + L - 1) // L

      def grp_body(g, _):
        dlv = cd_v[pl.ds(k * G + g * L, L)]
        dls = [dlv[t] for t in range(L)]
        for t in range(L):
          dl = dls[t]
          rvals = [rows_v[g * L + t, pl.ds(j * L, L)] for j in range(D // L)]
          avals = [agg_v[dl, pl.ds(j * L, L)] for j in range(D // L)]
          for j in range(D // L):
            agg_v[dl, pl.ds(j * L, L)] = jnp.maximum(avals[j], rvals[j])
        return 0

      lax.fori_loop(0, ngrp, grp_body, 0)
      return 0

    nb = (cnt + G - 1) // G
    lax.fori_loop(0, nb, blk_body, 0)
    return 0

  lax.fori_loop(0, E // CH, chunk_body, 0)

  pltpu.sync_copy(agg_v.at[pl.ds(0, NLOC)], out_hbm.at[pl.ds(lo, NLOC)])


_sc_mesh = plsc.VectorSubcoreMesh(core_axis_name="c", subcore_axis_name="s",
                                  num_cores=NC, num_subcores=NS)

_sc_call = functools.partial(
    pl.kernel,
    mesh=_sc_mesh,
    out_type=jax.ShapeDtypeStruct((NPAD, D), jnp.float32),
    scratch_types=[
        pltpu.VMEM((NLOC + 1, D), jnp.float32),
        pltpu.VMEM((CH,), jnp.int32),
        pltpu.VMEM((CH,), jnp.int32),
        pltpu.VMEM((CPAD,), jnp.int32),
        pltpu.VMEM((CPAD,), jnp.int32),
        pltpu.VMEM((G, D), jnp.float32),
        pltpu.SemaphoreType.DMA,
    ],
    compiler_params=pltpu.CompilerParams(needs_layout_passes=False),
)(_sc_body)


def _tc_body(agg_ref, x_ref, wl_ref, wr_ref, b_ref, out_ref):
  agg = agg_ref[...][:N]
  agg = jnp.where(jnp.isfinite(agg), agg, 0.0)
  logits = (
      jax.lax.dot_general(agg, wl_ref[...], (((1,), (0,)), ((), ())),
                          preferred_element_type=jnp.float32)
      + jax.lax.dot_general(x_ref[...], wr_ref[...], (((1,), (0,)), ((), ())),
                            preferred_element_type=jnp.float32)
      + b_ref[...]
  )
  m = jnp.max(logits, axis=1, keepdims=True)
  s = logits - m
  out_ref[...] = s - jnp.log(jnp.sum(jnp.exp(s), axis=1, keepdims=True))


_tc_call = pl.pallas_call(
    _tc_body,
    out_shape=jax.ShapeDtypeStruct((N, C), jnp.float32),
)


@jax.jit
def kernel(x, edge_index, W_l, b_l, W_r):
  src = edge_index[0]
  dst = edge_index[1]
  agg = _sc_call(x, src, dst)
  return _tc_call(agg, x, W_l.T, W_r.T, b_l.reshape(1, C))


# speedup vs baseline: 2.8097x; 1.2413x over previous
import functools

import jax
import jax.numpy as jnp
from jax import lax
from jax.experimental import pallas as pl
from jax.experimental.pallas import tpu as pltpu
from jax.experimental.pallas import tpu_sc as plsc

N = 10000
E = 320000
D = 128
C = 7

NC = 2
NS = 16
NW = NC * NS
L = 16

NLOC = 320
NPAD = NW * NLOC
CH = 3200
NCH = E // CH
G = 128
CPAD = CH + 2 * G
FB = 4
NEG = float("-inf")


def _sc_body(x_hbm, src_hbm, dst_hbm, out_hbm,
             agg_v, src_a, src_b, dst_a, dst_b, cs_a, cs_b, cd_a, cd_b,
             rows_a, rows_b, sem_e0, sem_e1, sem_g0, sem_g1):
  sem_e = (sem_e0, sem_e1)
  sem_g = (sem_g0, sem_g1)
  srcb = (src_a, src_b)
  dstb = (dst_a, dst_b)
  csb = (cs_a, cs_b)
  cdb = (cd_a, cd_b)
  rowsb = (rows_a, rows_b)

  wid = lax.axis_index("s") * NC + lax.axis_index("c")
  lo = wid * NLOC

  def start_edge(c, p):
    pltpu.async_copy(src_hbm.at[pl.ds(c * CH, CH)], srcb[p], sem_e[p])
    pltpu.async_copy(dst_hbm.at[pl.ds(c * CH, CH)], dstb[p], sem_e[p])

  def wait_edge(c, p):
    pltpu.make_async_copy(src_hbm.at[pl.ds(c * CH, CH)], srcb[p],
                          sem_e[p]).wait()
    pltpu.make_async_copy(dst_hbm.at[pl.ds(c * CH, CH)], dstb[p],
                          sem_e[p]).wait()

  def start_gather0(p):
    pltpu.async_copy(x_hbm.at[csb[p].at[pl.ds(0, G)]], rowsb[p],
                     sem_g[p])

  def wait_gather0(p):
    pltpu.make_async_copy(x_hbm.at[csb[p].at[pl.ds(0, G)]], rowsb[p],
                          sem_g[p]).wait()

  start_edge(0, 0)
  start_edge(1, 1)

  neg = jnp.full((L,), NEG, dtype=jnp.float32)

  def init_body(i, _):
    for j in range(D // L):
      agg_v[i, pl.ds(j * L, L)] = neg
    return 0

  lax.fori_loop(0, NLOC + 1, init_body, 0)

  lane = lax.iota(jnp.int32, L)
  lo_v = jnp.full((L,), lo, dtype=jnp.int32)
  nloc_u = jnp.full((L,), NLOC, dtype=jnp.uint32)
  nloc_i = jnp.full((L,), NLOC, dtype=jnp.int32)

  def do_filter(c, p):
    wait_edge(c, p)

    def filt_body(i, cnt_v):
      ss = [srcb[p][pl.ds((FB * i + k) * L, L)] for k in range(FB)]
      dd = [dstb[p][pl.ds((FB * i + k) * L, L)] for k in range(FB)]
      dls = [d - lo_v for d in dd]
      ms = [lax.bitcast_convert_type(dl, jnp.uint32) < nloc_u for dl in dls]
      mis = [m.astype(jnp.int32) for m in ms]
      ranks = [plsc.cumsum(mi) - mi for mi in mis]
      pops = [plsc.all_reduce_population_count(m) for m in ms]
      bases = [cnt_v]
      for k in range(FB - 1):
        bases.append(bases[k] + pops[k])
      for k in range(FB):
        pos = bases[k] + ranks[k]
        plsc.store_scatter(csb[p], [pos], ss[k], mask=ms[k])
        plsc.store_scatter(cdb[p], [pos], dls[k], mask=ms[k])
      return bases[FB - 1] + pops[FB - 1]

    cnt_v = lax.fori_loop(0, CH // (FB * L), filt_body,
                          jnp.zeros((L,), jnp.int32), unroll=2)
    cnt = lax.reduce_max(cnt_v, axes=(0,))

    f16 = (cnt // L) * L
    rem = cnt - f16
    keep = lane < rem
    tail_s = csb[p][pl.ds(f16, L)]
    csb[p][pl.ds(f16, L)] = jnp.where(keep, tail_s, lo_v)
    tail_d = cdb[p][pl.ds(f16, L)]
    cdb[p][pl.ds(f16, L)] = jnp.where(keep, tail_d, nloc_i)
    for t in range(G // L):
      csb[p][pl.ds(f16 + (t + 1) * L, L)] = lo_v
      cdb[p][pl.ds(f16 + (t + 1) * L, L)] = nloc_i
    return cnt

  def fold_block(p, k, cnt):
    ngrp = (jnp.minimum(cnt - k * G, G) + L - 1) // L

    def grp_body(g, _):
      dlv = cdb[p][pl.ds(k * G + g * L, L)]
      dls = [dlv[t] for t in range(L)]
      for t in range(L):
        dl = dls[t]
        rvals = [rowsb[p][g * L + t, pl.ds(j * L, L)]
                 for j in range(D // L)]
        avals = [agg_v[dl, pl.ds(j * L, L)] for j in range(D // L)]
        for j in range(D // L):
          agg_v[dl, pl.ds(j * L, L)] = jnp.maximum(avals[j], rvals[j])
      return 0

    lax.fori_loop(0, ngrp, grp_body, 0)

  def maxfold(p, cnt):
    fold_block(p, 0, cnt)
    nb = (cnt + G - 1) // G

    def extra(k, _):
      pltpu.async_copy(x_hbm.at[csb[p].at[pl.ds(k * G, G)]],
                       rowsb[p], sem_g[p]).wait()
      fold_block(p, k, cnt)
      return 0

    lax.fori_loop(1, nb, extra, 0)

  cnt0 = do_filter(0, 0)
  start_gather0(0)

  def body(c2, cnt):
    c = 2 * c2
    cnt_b = do_filter(c + 1, 1)
    start_edge(c + 2, 0)
    wait_gather0(0)
    start_gather0(1)
    maxfold(0, cnt)
    cnt_a = do_filter(c + 2, 0)
    start_edge(c + 3, 1)
    wait_gather0(1)
    start_gather0(0)
    maxfold(1, cnt_b)
    return cnt_a

  cnt_n2 = lax.fori_loop(0, (NCH - 2) // 2, body, cnt0)

  cnt_n1 = do_filter(NCH - 1, 1)
  wait_gather0(0)
  start_gather0(1)
  maxfold(0, cnt_n2)
  wait_gather0(1)
  maxfold(1, cnt_n1)

  pltpu.sync_copy(agg_v.at[pl.ds(0, NLOC)], out_hbm.at[pl.ds(lo, NLOC)])


_sc_mesh = plsc.VectorSubcoreMesh(core_axis_name="c", subcore_axis_name="s",
                                  num_cores=NC, num_subcores=NS)

_sc_call = functools.partial(
    pl.kernel,
    mesh=_sc_mesh,
    out_type=jax.ShapeDtypeStruct((NPAD, D), jnp.float32),
    scratch_types=[
        pltpu.VMEM((NLOC + 1, D), jnp.float32),
        pltpu.VMEM((CH,), jnp.int32),
        pltpu.VMEM((CH,), jnp.int32),
        pltpu.VMEM((CH,), jnp.int32),
        pltpu.VMEM((CH,), jnp.int32),
        pltpu.VMEM((CPAD,), jnp.int32),
        pltpu.VMEM((CPAD,), jnp.int32),
        pltpu.VMEM((CPAD,), jnp.int32),
        pltpu.VMEM((CPAD,), jnp.int32),
        pltpu.VMEM((G, D), jnp.float32),
        pltpu.VMEM((G, D), jnp.float32),
        pltpu.SemaphoreType.DMA,
        pltpu.SemaphoreType.DMA,
        pltpu.SemaphoreType.DMA,
        pltpu.SemaphoreType.DMA,
    ],
    compiler_params=pltpu.CompilerParams(needs_layout_passes=False),
)(_sc_body)


def _tc_body(agg_ref, x_ref, wl_ref, wr_ref, b_ref, out_ref):
  agg = agg_ref[...][:N]
  agg = jnp.where(jnp.isfinite(agg), agg, 0.0)
  logits = (
      jax.lax.dot_general(agg, wl_ref[...], (((1,), (0,)), ((), ())),
                          preferred_element_type=jnp.float32)
      + jax.lax.dot_general(x_ref[...], wr_ref[...], (((1,), (0,)), ((), ())),
                            preferred_element_type=jnp.float32)
      + b_ref[...]
  )
  m = jnp.max(logits, axis=1, keepdims=True)
  s = logits - m
  out_ref[...] = s - jnp.log(jnp.sum(jnp.exp(s), axis=1, keepdims=True))


_tc_call = pl.pallas_call(
    _tc_body,
    out_shape=jax.ShapeDtypeStruct((N, C), jnp.float32),
)


@jax.jit
def kernel(x, edge_index, W_l, b_l, W_r):
  src = edge_index[0]
  dst = edge_index[1]
  agg = _sc_call(x, src, dst)
  return _tc_call(agg, x, W_l.T, W_r.T, b_l.reshape(1, C))
